# Initial kernel scaffold; baseline (speedup 1.0000x reference)
import jax
import jax.numpy as jnp
from jax.experimental import pallas as pl


def kernel(x, edge_index, bn_gamma, bn_beta, W, b):
    raise NotImplementedError("write your pallas kernel here")


# speedup vs baseline: 11.6142x; 11.6142x over previous
import functools

import jax
import jax.numpy as jnp
from jax import lax
from jax.experimental import pallas as pl
from jax.experimental.pallas import tpu as pltpu
from jax.experimental.pallas import tpu_sc as plsc

_N = 50000
_E = 800000
_D = 48
_H = 64
_NP = 51200
_NR = 50048
_NSUB = 16
_NCORE = 2

_DEG_K = 40
_DEG_PER_TILE = _E // (_NCORE * _NSUB)
_DEG_NBLK = _DEG_PER_TILE // _DEG_K
_DEG_ZCHUNK = _NP // _NSUB


def _deg_body(dst_hbm, out_hbm, idx_v, ones_v, zbuf, deg_sh):
    c = lax.axis_index("c")
    s = lax.axis_index("s")
    wid = c * _NSUB + s

    zeros16 = jnp.zeros((16,), jnp.float32)
    ones16 = jnp.ones((16,), jnp.float32)

    def zfill(i, _):
        zbuf[pl.ds(i * 16, 16)] = zeros16
        return 0

    lax.fori_loop(0, _DEG_ZCHUNK // 16, zfill, 0)
    for j in range(3):
        ones_v[pl.ds(j * 16, 16)] = ones16

    pltpu.sync_copy(zbuf, deg_sh.at[pl.ds(s * _DEG_ZCHUNK, _DEG_ZCHUNK)])
    plsc.subcore_barrier()

    base = wid * _DEG_PER_TILE

    def blk(i, _):
        pltpu.sync_copy(dst_hbm.at[pl.ds(base + i * _DEG_K, _DEG_K)], idx_v)
        pltpu.sync_copy(ones_v.at[pl.ds(0, _DEG_K)], deg_sh.at[idx_v],
                        add=True)
        return 0

    lax.fori_loop(0, _DEG_NBLK, blk, 0)
    plsc.subcore_barrier()

    @pl.when(s == 0)
    def _():
        pltpu.sync_copy(deg_sh, out_hbm.at[pl.ds(c * _NP, _NP)])


def _deg_call(dst):
    mesh = plsc.VectorSubcoreMesh(core_axis_name="c", subcore_axis_name="s")
    fn = functools.partial(
        pl.kernel,
        mesh=mesh,
        out_type=jax.ShapeDtypeStruct((_NCORE * _NP,), jnp.float32),
        compiler_params=pltpu.CompilerParams(use_tc_tiling_on_sc=False),
        scratch_types=[
            pltpu.VMEM((_DEG_K,), jnp.int32),
            pltpu.VMEM((48,), jnp.float32),
            pltpu.VMEM((_DEG_ZCHUNK,), jnp.float32),
            pltpu.VMEM_SHARED((_NP,), jnp.float32),
        ],
    )(_deg_body)
    return fn(dst)


_MSG_K = 80
_MSG_PER_TILE = _E // _NSUB
_MSG_NBLK = _MSG_PER_TILE // _MSG_K
_MSG_ROWS_PER_TILE = _NR // _NSUB
_MSG_ZROWS = 136


def _msg_body(xs0_hbm, xs1_hbm, src_hbm, dst_hbm, acc0_hbm, acc1_hbm,
              sidx, didx, rows, zbuf, acc_sh, sem):
    c = lax.axis_index("c")
    s = lax.axis_index("s")

    zeros16 = jnp.zeros((16,), jnp.float32)

    def zfill(i, _):
        zbuf[i // 2, pl.ds((i % 2) * 16, 16)] = zeros16
        return 0

    lax.fori_loop(0, _MSG_ZROWS * 2, zfill, 0)

    zoff = s * _MSG_ROWS_PER_TILE
    for t in range(_MSG_ROWS_PER_TILE // _MSG_ZROWS):
        pltpu.sync_copy(zbuf, acc_sh.at[pl.ds(zoff + t * _MSG_ZROWS,
                                              _MSG_ZROWS)])
    plsc.subcore_barrier()

    ebase = s * _MSG_PER_TILE

    def blk(i, _):
        b = ebase + i * _MSG_K
        pltpu.sync_copy(src_hbm.at[pl.ds(b, _MSG_K)], sidx)
        pltpu.sync_copy(dst_hbm.at[pl.ds(b, _MSG_K)], didx)

        @pl.when(c == 0)
        def _():
            pltpu.async_copy(xs0_hbm.at[sidx], rows, sem).wait()

        @pl.when(c == 1)
        def _():
            pltpu.async_copy(xs1_hbm.at[sidx], rows, sem).wait()

        pltpu.sync_copy(rows, acc_sh.at[didx], add=True)
        return 0

    lax.fori_loop(0, _MSG_NBLK, blk, 0)
    plsc.subcore_barrier()

    r0 = s * _MSG_ROWS_PER_TILE

    @pl.when(c == 0)
    def _():
        pltpu.sync_copy(acc_sh.at[pl.ds(r0, _MSG_ROWS_PER_TILE)],
                        acc0_hbm.at[pl.ds(r0, _MSG_ROWS_PER_TILE)])

    @pl.when(c == 1)
    def _():
        pltpu.sync_copy(acc_sh.at[pl.ds(r0, _MSG_ROWS_PER_TILE)],
                        acc1_hbm.at[pl.ds(r0, _MSG_ROWS_PER_TILE)])


def _msg_call(xs0, xs1, src, dst):
    mesh = plsc.VectorSubcoreMesh(core_axis_name="c", subcore_axis_name="s")
    fn = functools.partial(
        pl.kernel,
        mesh=mesh,
        out_type=(jax.ShapeDtypeStruct((_NR, 32), jnp.float32),
                  jax.ShapeDtypeStruct((_NR, 32), jnp.float32)),
        compiler_params=pltpu.CompilerParams(use_tc_tiling_on_sc=False),
        scratch_types=[
            pltpu.VMEM((_MSG_K,), jnp.int32),
            pltpu.VMEM((_MSG_K,), jnp.int32),
            pltpu.VMEM((_MSG_K, 32), jnp.float32),
            pltpu.VMEM((_MSG_ZROWS, 32), jnp.float32),
            pltpu.VMEM_SHARED((_NR, 32), jnp.float32),
            pltpu.SemaphoreType.DMA,
        ],
    )(_msg_body)
    return fn(xs0, xs1, src, dst)


_BR = 2000


def _stats_body(x_ref, o_ref):
    i = pl.program_id(0)
    xb = x_ref[...]
    blk = jnp.concatenate(
        [jnp.sum(xb, axis=0, keepdims=True),
         jnp.sum(xb * xb, axis=0, keepdims=True)], axis=0)

    @pl.when(i == 0)
    def _():
        o_ref[...] = blk

    @pl.when(i > 0)
    def _():
        o_ref[...] += blk


def _stats_call(x):
    return pl.pallas_call(
        _stats_body,
        grid=(_N // _BR,),
        in_specs=[pl.BlockSpec((_BR, _D), lambda i: (i, 0))],
        out_specs=pl.BlockSpec((2, _D), lambda i: (0, 0)),
        out_shape=jax.ShapeDtypeStruct((2, _D), jnp.float32),
    )(x)


def _xform_body(x_ref, st_ref, g_ref, bt_ref, w_ref, d0_ref, d1_ref,
                o0_ref, o1_ref):
    st = st_ref[...]
    mean = st[0:1, :] * (1.0 / _N)
    var = st[1:2, :] * (1.0 / _N) - mean * mean
    rstd = lax.rsqrt(var + 1e-5)
    scale = g_ref[...] * rstd
    shift = bt_ref[...] - mean * scale
    xn = x_ref[...] * scale + shift
    xw = jnp.dot(xn, w_ref[...], preferred_element_type=jnp.float32)
    deg = d0_ref[...] + d1_ref[...] + 1.0
    dinv = lax.rsqrt(deg)
    xs = xw * dinv
    o0_ref[...] = xs[:, :32]
    o1_ref[...] = xs[:, 32:]


def _xform_call(x, st, g, bt, w, d0, d1):
    nb = _N // _BR
    return pl.pallas_call(
        _xform_body,
        grid=(nb,),
        in_specs=[
            pl.BlockSpec((_BR, _D), lambda i: (i, 0)),
            pl.BlockSpec((2, _D), lambda i: (0, 0)),
            pl.BlockSpec((1, _D), lambda i: (0, 0)),
            pl.BlockSpec((1, _D), lambda i: (0, 0)),
            pl.BlockSpec((_D, _H), lambda i: (0, 0)),
            pl.BlockSpec((_BR, 1), lambda i: (i, 0)),
            pl.BlockSpec((_BR, 1), lambda i: (i, 0)),
        ],
        out_specs=(pl.BlockSpec((_BR, 32), lambda i: (i, 0)),
                   pl.BlockSpec((_BR, 32), lambda i: (i, 0))),
        out_shape=(jax.ShapeDtypeStruct((_N, 32), jnp.float32),
                   jax.ShapeDtypeStruct((_N, 32), jnp.float32)),
    )(x, st, g, bt, w, d0, d1)


def _final_body(a0_ref, a1_ref, s0_ref, s1_ref, d0_ref, d1_ref, b_ref,
                o_ref):
    deg = d0_ref[...] + d1_ref[...] + 1.0
    dinv = lax.rsqrt(deg)
    left = (a0_ref[...] + s0_ref[...]) * dinv
    right = (a1_ref[...] + s1_ref[...]) * dinv
    pre = jnp.concatenate([left, right], axis=1) + b_ref[...]
    o_ref[...] = jnp.maximum(pre, 0.0)


def _final_call(a0, a1, s0, s1, d0, d1, b):
    nb = _N // _BR
    return pl.pallas_call(
        _final_body,
        grid=(nb,),
        in_specs=[
            pl.BlockSpec((_BR, 32), lambda i: (i, 0)),
            pl.BlockSpec((_BR, 32), lambda i: (i, 0)),
            pl.BlockSpec((_BR, 32), lambda i: (i, 0)),
            pl.BlockSpec((_BR, 32), lambda i: (i, 0)),
            pl.BlockSpec((_BR, 1), lambda i: (i, 0)),
            pl.BlockSpec((_BR, 1), lambda i: (i, 0)),
            pl.BlockSpec((1, _H), lambda i: (0, 0)),
        ],
        out_specs=pl.BlockSpec((_BR, _H), lambda i: (i, 0)),
        out_shape=jax.ShapeDtypeStruct((_N, _H), jnp.float32),
    )(a0, a1, s0, s1, d0, d1, b)


def kernel(x, edge_index, bn_gamma, bn_beta, W, b):
    src = edge_index[0]
    dst = edge_index[1]

    degp = _deg_call(dst)
    st = _stats_call(x)

    d0 = degp[:_N].reshape(_N, 1)
    d1 = degp[_NP:_NP + _N].reshape(_N, 1)

    xs0, xs1 = _xform_call(x, st, bn_gamma.reshape(1, _D),
                           bn_beta.reshape(1, _D), W, d0, d1)
    acc0, acc1 = _msg_call(xs0, xs1, src, dst)
    return _final_call(acc0, acc1, xs0, xs1, d0, d1, b.reshape(1, _H))


# speedup vs baseline: 39.4469x; 3.3964x over previous
import functools

import jax
import jax.numpy as jnp
from jax import lax
from jax.experimental import pallas as pl
from jax.experimental.pallas import tpu as pltpu
from jax.experimental.pallas import tpu_sc as plsc

_N = 50000
_E = 800000
_D = 48
_H = 64
_NP = 51200
_NR = 50048
_NSUB = 16
_NCORE = 2
_EP = 802816

_DEG_K = 64
_DEG_NBLK = _EP // (_NCORE * _NSUB * _DEG_K)
_DEG_ZCHUNK = _NP // _NSUB


def _deg_body(dst_hbm, out_hbm, didx_all, ones_v, zbuf, deg_sh, sem):
    c = lax.axis_index("c")
    s = lax.axis_index("s")
    wid = c * _NSUB + s

    zeros16 = jnp.zeros((16,), jnp.float32)
    ones16 = jnp.ones((16,), jnp.float32)

    def zfill(i, _):
        zbuf[pl.ds(i * 16, 16)] = zeros16
        return 0

    lax.fori_loop(0, _DEG_ZCHUNK // 16, zfill, 0)
    for j in range(_DEG_K // 16):
        ones_v[pl.ds(j * 16, 16)] = ones16

    pltpu.sync_copy(zbuf, deg_sh.at[pl.ds(s * _DEG_ZCHUNK, _DEG_ZCHUNK)])
    pltpu.sync_copy(dst_hbm.at[pl.ds(wid * _DEG_NBLK, _DEG_NBLK)], didx_all)
    plsc.subcore_barrier()

    def fire(j, _):
        pltpu.async_copy(ones_v, deg_sh.at[didx_all.at[j]], sem, add=True)
        return 0

    lax.fori_loop(0, _DEG_NBLK, fire, 0)

    def drain(j, _):
        pltpu.make_async_copy(ones_v, deg_sh.at[didx_all.at[j]], sem).wait()
        return 0

    lax.fori_loop(0, _DEG_NBLK, drain, 0)
    plsc.subcore_barrier()

    @pl.when(s == 0)
    def _():
        pltpu.sync_copy(deg_sh, out_hbm.at[pl.ds(c * _NP, _NP)])


def _deg_call(dst2d):
    mesh = plsc.VectorSubcoreMesh(core_axis_name="c", subcore_axis_name="s")
    fn = functools.partial(
        pl.kernel,
        mesh=mesh,
        out_type=jax.ShapeDtypeStruct((_NCORE * _NP,), jnp.float32),
        compiler_params=pltpu.CompilerParams(use_tc_tiling_on_sc=False),
        scratch_types=[
            pltpu.VMEM((_DEG_NBLK, _DEG_K), jnp.int32),
            pltpu.VMEM((_DEG_K,), jnp.float32),
            pltpu.VMEM((_DEG_ZCHUNK,), jnp.float32),
            pltpu.VMEM_SHARED((_NP,), jnp.float32),
            pltpu.SemaphoreType.DMA,
        ],
    )(_deg_body)
    return fn(dst2d)


_MSG_K = 128
_MSG_NBLK = _EP // (_NSUB * _MSG_K)
_MSG_ROWS_PER_TILE = _NR // _NSUB
_MSG_DEPTH = 4
_MSG_SUP = 8
_MSG_NSUP = _MSG_NBLK // _MSG_SUP


def _msg_body(xs0_hbm, xs1_hbm, src_hbm, dst_hbm, acc0_hbm, acc1_hbm,
              sidx_a, didx_a, sidx_b, didx_b, rows0, rows1, rows2, rows3,
              acc_sh, isem, sem0, sem1, sem2, sem3):
    c = lax.axis_index("c")
    s = lax.axis_index("s")
    rows = (rows0, rows1, rows2, rows3)
    sems = (sem0, sem1, sem2, sem3)

    zeros16 = jnp.zeros((16,), jnp.float32)

    def zfill(i, _):
        rows0[i // 2, pl.ds((i % 2) * 16, 16)] = zeros16
        return 0

    lax.fori_loop(0, _MSG_K * 2, zfill, 0)

    zoff = s * _MSG_ROWS_PER_TILE
    for t in range(24):
        pltpu.sync_copy(rows0, acc_sh.at[pl.ds(zoff + t * _MSG_K,
                                               _MSG_K)])
    pltpu.sync_copy(rows0.at[pl.ds(0, 56)],
                    acc_sh.at[pl.ds(zoff + 24 * _MSG_K, 56)])

    ebase = s * _MSG_NBLK

    def idx_fire(g, sb, db):
        pltpu.async_copy(src_hbm.at[pl.ds(ebase + g * _MSG_SUP, _MSG_SUP)],
                         sb, isem)
        pltpu.async_copy(dst_hbm.at[pl.ds(ebase + g * _MSG_SUP, _MSG_SUP)],
                         db, isem)

    def idx_wait(g, sb, db):
        pltpu.make_async_copy(
            src_hbm.at[pl.ds(ebase + g * _MSG_SUP, _MSG_SUP)], sb,
            isem).wait()
        pltpu.make_async_copy(
            dst_hbm.at[pl.ds(ebase + g * _MSG_SUP, _MSG_SUP)], db,
            isem).wait()

    def gather(idx_row, r):
        @pl.when(c == 0)
        def _():
            pltpu.async_copy(xs0_hbm.at[idx_row], rows[r], sems[r])

        @pl.when(c == 1)
        def _():
            pltpu.async_copy(xs1_hbm.at[idx_row], rows[r], sems[r])

    def gather_wait(idx_row, r):
        @pl.when(c == 0)
        def _():
            pltpu.make_async_copy(xs0_hbm.at[idx_row], rows[r],
                                  sems[r]).wait()

        @pl.when(c == 1)
        def _():
            pltpu.make_async_copy(xs1_hbm.at[idx_row], rows[r],
                                  sems[r]).wait()

    idx_fire(0, sidx_a, didx_a)
    idx_wait(0, sidx_a, didx_a)
    plsc.subcore_barrier()
    for t in range(_MSG_DEPTH):
        gather(sidx_a.at[t], t)

    def superblock(g, sidx_cur, didx_cur, sidx_nxt, didx_nxt):
        @pl.when(g + 1 < _MSG_NSUP)
        def _():
            idx_fire(g + 1, sidx_nxt, didx_nxt)

        for t in range(_MSG_SUP):
            if t == _MSG_DEPTH:
                @pl.when(g + 1 < _MSG_NSUP)
                def _():
                    idx_wait(g + 1, sidx_nxt, didx_nxt)
            r = t % _MSG_DEPTH
            gather_wait(sidx_cur.at[t], r)
            pltpu.sync_copy(rows[r], acc_sh.at[didx_cur.at[t]], add=True)
            if t < _MSG_DEPTH:
                gather(sidx_cur.at[t + _MSG_DEPTH], r)
            else:
                @pl.when(g + 1 < _MSG_NSUP)
                def _(t=t, r=r):
                    gather(sidx_nxt.at[t - _MSG_DEPTH], r)

    def sup_step(g, _):
        @pl.when(lax.rem(g, 2) == 0)
        def _():
            superblock(g, sidx_a, didx_a, sidx_b, didx_b)

        @pl.when(lax.rem(g, 2) == 1)
        def _():
            superblock(g, sidx_b, didx_b, sidx_a, didx_a)

        return 0

    lax.fori_loop(0, _MSG_NSUP, sup_step, 0)
    plsc.subcore_barrier()

    r0 = s * _MSG_ROWS_PER_TILE

    @pl.when(c == 0)
    def _():
        pltpu.sync_copy(acc_sh.at[pl.ds(r0, _MSG_ROWS_PER_TILE)],
                        acc0_hbm.at[pl.ds(r0, _MSG_ROWS_PER_TILE)])

    @pl.when(c == 1)
    def _():
        pltpu.sync_copy(acc_sh.at[pl.ds(r0, _MSG_ROWS_PER_TILE)],
                        acc1_hbm.at[pl.ds(r0, _MSG_ROWS_PER_TILE)])


def _msg_call(xs0, xs1, src2d, dst2d):
    mesh = plsc.VectorSubcoreMesh(core_axis_name="c", subcore_axis_name="s")
    fn = functools.partial(
        pl.kernel,
        mesh=mesh,
        out_type=(jax.ShapeDtypeStruct((_NR, 32), jnp.float32),
                  jax.ShapeDtypeStruct((_NR, 32), jnp.float32)),
        compiler_params=pltpu.CompilerParams(use_tc_tiling_on_sc=False),
        scratch_types=[
            pltpu.VMEM((_MSG_SUP, _MSG_K), jnp.int32),
            pltpu.VMEM((_MSG_SUP, _MSG_K), jnp.int32),
            pltpu.VMEM((_MSG_SUP, _MSG_K), jnp.int32),
            pltpu.VMEM((_MSG_SUP, _MSG_K), jnp.int32),
            pltpu.VMEM((_MSG_K, 32), jnp.float32),
            pltpu.VMEM((_MSG_K, 32), jnp.float32),
            pltpu.VMEM((_MSG_K, 32), jnp.float32),
            pltpu.VMEM((_MSG_K, 32), jnp.float32),
            pltpu.VMEM_SHARED((_NR, 32), jnp.float32),
            pltpu.SemaphoreType.DMA,
            pltpu.SemaphoreType.DMA,
            pltpu.SemaphoreType.DMA,
            pltpu.SemaphoreType.DMA,
            pltpu.SemaphoreType.DMA,
        ],
    )(_msg_body)
    return fn(xs0, xs1, src2d, dst2d)


_BR = 2000


def _stats_body(x_ref, o_ref):
    i = pl.program_id(0)
    xb = x_ref[...]
    blk = jnp.concatenate(
        [jnp.sum(xb, axis=0, keepdims=True),
         jnp.sum(xb * xb, axis=0, keepdims=True)], axis=0)

    @pl.when(i == 0)
    def _():
        o_ref[...] = blk

    @pl.when(i > 0)
    def _():
        o_ref[...] += blk


def _stats_call(x):
    return pl.pallas_call(
        _stats_body,
        grid=(_N // _BR,),
        in_specs=[pl.BlockSpec((_BR, _D), lambda i: (i, 0))],
        out_specs=pl.BlockSpec((2, _D), lambda i: (0, 0)),
        out_shape=jax.ShapeDtypeStruct((2, _D), jnp.float32),
    )(x)


def _xform_body(x_ref, st_ref, g_ref, bt_ref, w_ref, d0_ref, d1_ref,
                o0_ref, o1_ref):
    st = st_ref[...]
    mean = st[0:1, :] * (1.0 / _N)
    var = st[1:2, :] * (1.0 / _N) - mean * mean
    rstd = lax.rsqrt(var + 1e-5)
    scale = g_ref[...] * rstd
    shift = bt_ref[...] - mean * scale
    xn = x_ref[...] * scale + shift
    xw = jnp.dot(xn, w_ref[...], preferred_element_type=jnp.float32)
    deg = d0_ref[...] + d1_ref[...] + 1.0
    dinv = lax.rsqrt(deg)
    xs = xw * dinv
    o0_ref[...] = xs[:, :32]
    o1_ref[...] = xs[:, 32:]


def _xform_call(x, st, g, bt, w, d0, d1):
    nb = _N // _BR
    return pl.pallas_call(
        _xform_body,
        grid=(nb,),
        in_specs=[
            pl.BlockSpec((_BR, _D), lambda i: (i, 0)),
            pl.BlockSpec((2, _D), lambda i: (0, 0)),
            pl.BlockSpec((1, _D), lambda i: (0, 0)),
            pl.BlockSpec((1, _D), lambda i: (0, 0)),
            pl.BlockSpec((_D, _H), lambda i: (0, 0)),
            pl.BlockSpec((_BR, 1), lambda i: (i, 0)),
            pl.BlockSpec((_BR, 1), lambda i: (i, 0)),
        ],
        out_specs=(pl.BlockSpec((_BR, 32), lambda i: (i, 0)),
                   pl.BlockSpec((_BR, 32), lambda i: (i, 0))),
        out_shape=(jax.ShapeDtypeStruct((_N, 32), jnp.float32),
                   jax.ShapeDtypeStruct((_N, 32), jnp.float32)),
    )(x, st, g, bt, w, d0, d1)


def _final_body(a0_ref, a1_ref, s0_ref, s1_ref, d0_ref, d1_ref, b_ref,
                o_ref):
    deg = d0_ref[...] + d1_ref[...] + 1.0
    dinv = lax.rsqrt(deg)
    left = (a0_ref[...] + s0_ref[...]) * dinv
    right = (a1_ref[...] + s1_ref[...]) * dinv
    pre = jnp.concatenate([left, right], axis=1) + b_ref[...]
    o_ref[...] = jnp.maximum(pre, 0.0)


def _final_call(a0, a1, s0, s1, d0, d1, b):
    nb = _N // _BR
    return pl.pallas_call(
        _final_body,
        grid=(nb,),
        in_specs=[
            pl.BlockSpec((_BR, 32), lambda i: (i, 0)),
            pl.BlockSpec((_BR, 32), lambda i: (i, 0)),
            pl.BlockSpec((_BR, 32), lambda i: (i, 0)),
            pl.BlockSpec((_BR, 32), lambda i: (i, 0)),
            pl.BlockSpec((_BR, 1), lambda i: (i, 0)),
            pl.BlockSpec((_BR, 1), lambda i: (i, 0)),
            pl.BlockSpec((1, _H), lambda i: (0, 0)),
        ],
        out_specs=pl.BlockSpec((_BR, _H), lambda i: (i, 0)),
        out_shape=jax.ShapeDtypeStruct((_N, _H), jnp.float32),
    )(a0, a1, s0, s1, d0, d1, b)


def kernel(x, edge_index, bn_gamma, bn_beta, W, b):
    npad = _EP - _E
    src = jnp.concatenate(
        [edge_index[0], jnp.zeros((npad,), jnp.int32)])
    dst = jnp.concatenate(
        [edge_index[1], jnp.full((npad,), _N, jnp.int32)])
    src_msg = src.reshape(_NSUB * _MSG_NBLK, _MSG_K)
    dst_msg = dst.reshape(_NSUB * _MSG_NBLK, _MSG_K)
    dst_deg = dst.reshape(_NCORE * _NSUB * _DEG_NBLK, _DEG_K)

    degp = _deg_call(dst_deg)
    st = _stats_call(x)

    d0 = degp[:_N].reshape(_N, 1)
    d1 = degp[_NP:_NP + _N].reshape(_N, 1)

    xs0, xs1 = _xform_call(x, st, bn_gamma.reshape(1, _D),
                           bn_beta.reshape(1, _D), W, d0, d1)
    acc0, acc1 = _msg_call(xs0, xs1, src_msg, dst_msg)
    return _final_call(acc0, acc1, xs0, xs1, d0, d1, b.reshape(1, _H))
